# baseline (speedup 1.0000x reference)
import jax
import jax.numpy as jnp
from jax.experimental import pallas as pl
import jax.experimental.pallas.tpu as pltpu

M_TILE = 512
N_CHUNKS = 8


def _gcn_kernel(x_ref, a_ref, w_ref, o_ref, xb_ref, wb_ref, r_ref):
    i = pl.program_id(0)
    chunk = a_ref.shape[1]

    @pl.when(i == 0)
    def _():
        xb_ref[:] = x_ref[:].astype(jnp.bfloat16)
        wb_ref[:] = w_ref[:].astype(jnp.bfloat16)

    @pl.when(i < N_CHUNKS)
    def _():
        wa = jnp.dot(
            wb_ref[:],
            a_ref[:].astype(jnp.bfloat16),
            preferred_element_type=jnp.float32,
        )
        r_ref[:, pl.ds(i * chunk, chunk)] = jnp.dot(
            xb_ref[:],
            wa.astype(jnp.bfloat16),
            preferred_element_type=jnp.float32,
        )

    @pl.when(i >= N_CHUNKS)
    def _():
        t = i - N_CHUNKS
        e = jnp.exp(r_ref[pl.ds(t * M_TILE, M_TILE), :])
        o_ref[:] = e * (1.0 / jnp.sum(e, axis=-1, keepdims=True))


def kernel(inputs, normalized_adjacency, weights):
    n, din = inputs.shape
    dout = weights.shape[1]
    n_row_tiles = n // M_TILE
    grid = (N_CHUNKS + n_row_tiles,)
    return pl.pallas_call(
        _gcn_kernel,
        grid=grid,
        in_specs=[
            pl.BlockSpec((n, din), lambda i: (0, 0)),
            pl.BlockSpec(
                (dout, n // N_CHUNKS),
                lambda i: (0, jnp.minimum(i, N_CHUNKS - 1)),
            ),
            pl.BlockSpec((din, dout), lambda i: (0, 0)),
        ],
        out_specs=pl.BlockSpec(
            (M_TILE, n),
            lambda i: (jnp.where(i < N_CHUNKS, 0, i - N_CHUNKS), 0),
        ),
        out_shape=jax.ShapeDtypeStruct((n, normalized_adjacency.shape[0]), jnp.float32),
        scratch_shapes=[
            pltpu.VMEM((n, din), jnp.bfloat16),
            pltpu.VMEM((din, dout), jnp.bfloat16),
            pltpu.VMEM((n, n), jnp.float32),
        ],
    )(inputs, normalized_adjacency, weights)


# speedup vs baseline: 1.3963x; 1.3963x over previous
import jax
import jax.numpy as jnp
from jax.experimental import pallas as pl
import jax.experimental.pallas.tpu as pltpu

M_TILE = 512
N_CHUNKS = 8


def _gcn_kernel(x_ref, a_ref, w_ref, o_ref, wa_ref):
    i = pl.program_id(0)
    chunk = a_ref.shape[1]

    @pl.when(i < N_CHUNKS)
    def _():
        wa = jnp.dot(
            w_ref[:].astype(jnp.bfloat16),
            a_ref[:].astype(jnp.bfloat16),
            preferred_element_type=jnp.float32,
        )
        wa_ref[:, pl.ds(i * chunk, chunk)] = wa.astype(jnp.bfloat16)

    @pl.when(i >= N_CHUNKS)
    def _():
        r = jnp.dot(
            x_ref[:].astype(jnp.bfloat16),
            wa_ref[:],
            preferred_element_type=jnp.float32,
        )
        e = jnp.exp(r)
        o_ref[:] = e * (1.0 / jnp.sum(e, axis=-1, keepdims=True))


def kernel(inputs, normalized_adjacency, weights):
    n, din = inputs.shape
    dout = weights.shape[1]
    n_row_tiles = n // M_TILE
    grid = (N_CHUNKS + n_row_tiles,)
    return pl.pallas_call(
        _gcn_kernel,
        grid=grid,
        in_specs=[
            pl.BlockSpec(
                (M_TILE, din),
                lambda i: (jnp.where(i < N_CHUNKS, 0, i - N_CHUNKS), 0),
            ),
            pl.BlockSpec(
                (dout, n // N_CHUNKS),
                lambda i: (0, jnp.minimum(i, N_CHUNKS - 1)),
            ),
            pl.BlockSpec((din, dout), lambda i: (0, 0)),
        ],
        out_specs=pl.BlockSpec(
            (M_TILE, n),
            lambda i: (jnp.where(i < N_CHUNKS, 0, i - N_CHUNKS), 0),
        ),
        out_shape=jax.ShapeDtypeStruct((n, normalized_adjacency.shape[0]), jnp.float32),
        scratch_shapes=[pltpu.VMEM((din, n), jnp.bfloat16)],
    )(inputs, normalized_adjacency, weights)


# speedup vs baseline: 1.4115x; 1.0109x over previous
import jax
import jax.numpy as jnp
from jax.experimental import pallas as pl
import jax.experimental.pallas.tpu as pltpu

M_TILE = 512


def _gcn_kernel(x_ref, a_ref, w_ref, o_ref, wa_ref):
    @pl.when(pl.program_id(0) == 0)
    def _():
        wa = jnp.dot(
            w_ref[:].astype(jnp.bfloat16),
            a_ref[:].astype(jnp.bfloat16),
            preferred_element_type=jnp.float32,
        )
        wa_ref[:] = wa.astype(jnp.bfloat16)

    r = jnp.dot(
        x_ref[:].astype(jnp.bfloat16),
        wa_ref[:],
        preferred_element_type=jnp.float32,
    )
    e = jnp.exp(r)
    o_ref[:] = e * (1.0 / jnp.sum(e, axis=-1, keepdims=True))


def kernel(inputs, normalized_adjacency, weights):
    n, din = inputs.shape
    dout = weights.shape[1]
    grid = (n // M_TILE,)
    return pl.pallas_call(
        _gcn_kernel,
        grid=grid,
        in_specs=[
            pl.BlockSpec((M_TILE, din), lambda i: (i, 0)),
            pl.BlockSpec((dout, n), lambda i: (0, 0)),
            pl.BlockSpec((din, dout), lambda i: (0, 0)),
        ],
        out_specs=pl.BlockSpec((M_TILE, n), lambda i: (i, 0)),
        out_shape=jax.ShapeDtypeStruct((n, normalized_adjacency.shape[0]), jnp.float32),
        scratch_shapes=[pltpu.VMEM((din, n), jnp.bfloat16)],
    )(inputs, normalized_adjacency, weights)
